# baseline (speedup 1.0000x reference)
import jax
import jax.numpy as jnp
from jax import lax
from jax.experimental import pallas as pl
from jax.experimental.pallas import tpu as pltpu
from jax.experimental.pallas import tpu_sc as plsc

_NW = 32
_CHUNK = 10000
_VEC = _CHUNK // 16
_MAXK = 13

_PHI_N = 200
_PHI2_N = 100
_PHI_PAD = 208
_PHI2_PAD = 112
_HIDDEN = 3
_SCALE1 = float(_PHI_N - 1)
_SCALE2 = float(_PHI2_N - 1) / 6.0


def _sc_body(x_hbm, phi_hbm, big_hbm, par_hbm, out_hbm,
             xb0, xb1, ob0, ob1, phib, bigb, parb,
             isem0, isem1, osem0, osem1):
    nchunks = x_hbm.shape[0] // _CHUNK
    wid = lax.axis_index("s") * 2 + lax.axis_index("c")
    nk = jnp.where(wid < nchunks % _NW, _MAXK, _MAXK - 1)

    pltpu.sync_copy(phi_hbm, phib)
    pltpu.sync_copy(big_hbm, bigb)
    pltpu.sync_copy(par_hbm, parb)
    eta_v = parb[pl.ds(0, 16)]
    lam_v = parb[pl.ds(16, 16)]
    shift = [eta_v * (_SCALE1 * q) for q in range(_HIDDEN)]
    lam2 = lam_v * _SCALE2

    xbufs, obufs = (xb0, xb1), (ob0, ob1)
    isems, osems = (isem0, isem1), (osem0, osem1)

    def start_in(k, b):
        off = (wid + _NW * k) * _CHUNK
        pltpu.async_copy(x_hbm.at[pl.ds(off, _CHUNK)], xbufs[b], isems[b])

    def wait_in(b):
        pltpu.make_async_copy(
            x_hbm.at[pl.ds(0, _CHUNK)], xbufs[b], isems[b]).wait()

    def start_out(k, b):
        off = (wid + _NW * k) * _CHUNK
        pltpu.async_copy(obufs[b], out_hbm.at[pl.ds(off, _CHUNK)], osems[b])

    def wait_out(b):
        pltpu.make_async_copy(
            obufs[b], out_hbm.at[pl.ds(0, _CHUNK)], osems[b]).wait()

    def compute(b):
        xb, ob = xbufs[b], obufs[b]

        @plsc.parallel_loop(0, _VEC, unroll=10)
        def _vec(i):
            v = xb[pl.ds(i * 16, 16)]
            acc = None
            for q in range(_HIDDEN):
                f = jnp.minimum(v * _SCALE1 + shift[q], _SCALE1)
                ii = jnp.minimum(f.astype(jnp.int32), _PHI_N - 2)
                t = f - ii.astype(jnp.float32)
                c0 = plsc.load_gather(phib, [ii])
                c1 = plsc.load_gather(phib, [ii + 1])
                phi = c0 + t * (c1 - c0)
                f2 = jnp.clip(phi * lam2 + (_SCALE2 * (q + 3.0)),
                              0.0, 6.0 * _SCALE2)
                jj = jnp.minimum(f2.astype(jnp.int32), _PHI2_N - 2)
                t2 = f2 - jj.astype(jnp.float32)
                d0 = plsc.load_gather(bigb, [jj])
                d1 = plsc.load_gather(bigb, [jj + 1])
                r = d0 + t2 * (d1 - d0)
                acc = r if acc is None else acc + r
            ob[pl.ds(i * 16, 16)] = acc

    start_in(0, 0)
    start_in(1, 1)

    @pl.loop(0, _MAXK - 1, step=2)
    def _pair(k):
        for b in range(2):
            kk = k + b
            wait_in(b)

            @pl.when(kk >= 2)
            def _drain():
                wait_out(b)

            compute(b)
            start_out(kk, b)

            @pl.when(kk + 2 < nk)
            def _next():
                start_in(kk + 2, b)

    @pl.when(nk == _MAXK)
    def _tail():
        wait_in(0)
        wait_out(0)
        compute(0)
        start_out(_MAXK - 1, 0)

    wait_out(0)
    wait_out(1)


def _make_sc_kernel(n):
    mesh = plsc.VectorSubcoreMesh(core_axis_name="c", subcore_axis_name="s")
    return pl.kernel(
        _sc_body,
        mesh=mesh,
        compiler_params=pltpu.CompilerParams(needs_layout_passes=False),
        out_type=jax.ShapeDtypeStruct((n,), jnp.float32),
        scratch_types=[
            pltpu.VMEM((_CHUNK,), jnp.float32),
            pltpu.VMEM((_CHUNK,), jnp.float32),
            pltpu.VMEM((_CHUNK,), jnp.float32),
            pltpu.VMEM((_CHUNK,), jnp.float32),
            pltpu.VMEM((_PHI_PAD,), jnp.float32),
            pltpu.VMEM((_PHI2_PAD,), jnp.float32),
            pltpu.VMEM((32,), jnp.float32),
            pltpu.SemaphoreType.DMA,
            pltpu.SemaphoreType.DMA,
            pltpu.SemaphoreType.DMA,
            pltpu.SemaphoreType.DMA,
        ],
    )


def kernel(x, phi_coeffs, Phi_coeffs, lambdas, eta):
    n = x.shape[0]
    xf = x.reshape(n)
    phi_p = jnp.zeros((_PHI_PAD,), jnp.float32).at[:_PHI_N].set(phi_coeffs)
    big_p = jnp.zeros((_PHI2_PAD,), jnp.float32).at[:_PHI2_N].set(Phi_coeffs)
    par = jnp.concatenate([
        jnp.full((16,), eta, jnp.float32),
        jnp.full((16,), lambdas[0], jnp.float32),
    ])
    out = _make_sc_kernel(n)(xf, phi_p, big_p, par)
    return out.reshape(n, 1)


# speedup vs baseline: 1.0820x; 1.0820x over previous
import jax
import jax.numpy as jnp
from jax import lax
from jax.experimental import pallas as pl
from jax.experimental.pallas import tpu as pltpu
from jax.experimental.pallas import tpu_sc as plsc

_NW = 32
_CHUNK = 10000
_VEC = _CHUNK // 16
_MAXK = 13

_PHI_N = 200
_PHI2_N = 100
_PHI_PAD = 208
_PHI2_PAD = 112
_HIDDEN = 3
_SCALE1 = float(_PHI_N - 1)
_SCALE2 = float(_PHI2_N - 1) / 6.0


def _sc_body(x_hbm, phi_hbm, big_hbm, par_hbm, out_hbm,
             xb0, xb1, ob0, ob1, phib, bigb, parb,
             isem0, isem1, osem0, osem1):
    nchunks = x_hbm.shape[0] // _CHUNK
    wid = lax.axis_index("s") * 2 + lax.axis_index("c")
    nk = jnp.where(wid < nchunks % _NW, _MAXK, _MAXK - 1)

    pltpu.sync_copy(phi_hbm, phib)
    pltpu.sync_copy(big_hbm, bigb)
    pltpu.sync_copy(par_hbm, parb)
    eta_v = parb[pl.ds(0, 16)]
    lam_v = parb[pl.ds(16, 16)]
    shift = [eta_v * (_SCALE1 * q) for q in range(_HIDDEN)]
    lam2 = lam_v * _SCALE2

    xbufs, obufs = (xb0, xb1), (ob0, ob1)
    isems, osems = (isem0, isem1), (osem0, osem1)

    def start_in(k, b):
        off = (wid + _NW * k) * _CHUNK
        pltpu.async_copy(x_hbm.at[pl.ds(off, _CHUNK)], xbufs[b], isems[b])

    def wait_in(b):
        pltpu.make_async_copy(
            x_hbm.at[pl.ds(0, _CHUNK)], xbufs[b], isems[b]).wait()

    def start_out(k, b):
        off = (wid + _NW * k) * _CHUNK
        pltpu.async_copy(obufs[b], out_hbm.at[pl.ds(off, _CHUNK)], osems[b])

    def wait_out(b):
        pltpu.make_async_copy(
            obufs[b], out_hbm.at[pl.ds(0, _CHUNK)], osems[b]).wait()

    def compute(b):
        xb, ob = xbufs[b], obufs[b]

        @plsc.parallel_loop(0, _VEC, unroll=5)
        def _vec(i):
            v = xb[pl.ds(i * 16, 16)]
            acc = None
            for q in range(_HIDDEN):
                f = jnp.minimum(v * _SCALE1 + shift[q], _SCALE1)
                ii = jnp.minimum(f.astype(jnp.int32), _PHI_N - 2)
                t = f - ii.astype(jnp.float32)
                c0 = plsc.load_gather(phib, [ii])
                c1 = plsc.load_gather(phib, [ii + 1])
                phi = c0 + t * (c1 - c0)
                f2 = jnp.clip(phi * lam2 + (_SCALE2 * (q + 3.0)),
                              0.0, 6.0 * _SCALE2)
                jj = jnp.minimum(f2.astype(jnp.int32), _PHI2_N - 2)
                t2 = f2 - jj.astype(jnp.float32)
                d0 = plsc.load_gather(bigb, [jj])
                d1 = plsc.load_gather(bigb, [jj + 1])
                r = d0 + t2 * (d1 - d0)
                acc = r if acc is None else acc + r
            ob[pl.ds(i * 16, 16)] = acc

    start_in(0, 0)
    start_in(1, 1)

    @pl.loop(0, _MAXK - 1, step=2)
    def _pair(k):
        for b in range(2):
            kk = k + b
            wait_in(b)

            @pl.when(kk >= 2)
            def _drain():
                wait_out(b)

            compute(b)
            start_out(kk, b)

            @pl.when(kk + 2 < nk)
            def _next():
                start_in(kk + 2, b)

    @pl.when(nk == _MAXK)
    def _tail():
        wait_in(0)
        wait_out(0)
        compute(0)
        start_out(_MAXK - 1, 0)

    wait_out(0)
    wait_out(1)


def _make_sc_kernel(n):
    mesh = plsc.VectorSubcoreMesh(core_axis_name="c", subcore_axis_name="s")
    return pl.kernel(
        _sc_body,
        mesh=mesh,
        compiler_params=pltpu.CompilerParams(needs_layout_passes=False),
        out_type=jax.ShapeDtypeStruct((n,), jnp.float32),
        scratch_types=[
            pltpu.VMEM((_CHUNK,), jnp.float32),
            pltpu.VMEM((_CHUNK,), jnp.float32),
            pltpu.VMEM((_CHUNK,), jnp.float32),
            pltpu.VMEM((_CHUNK,), jnp.float32),
            pltpu.VMEM((_PHI_PAD,), jnp.float32),
            pltpu.VMEM((_PHI2_PAD,), jnp.float32),
            pltpu.VMEM((32,), jnp.float32),
            pltpu.SemaphoreType.DMA,
            pltpu.SemaphoreType.DMA,
            pltpu.SemaphoreType.DMA,
            pltpu.SemaphoreType.DMA,
        ],
    )


def kernel(x, phi_coeffs, Phi_coeffs, lambdas, eta):
    n = x.shape[0]
    xf = x.reshape(n)
    phi_p = jnp.zeros((_PHI_PAD,), jnp.float32).at[:_PHI_N].set(phi_coeffs)
    big_p = jnp.zeros((_PHI2_PAD,), jnp.float32).at[:_PHI2_N].set(Phi_coeffs)
    par = jnp.concatenate([
        jnp.full((16,), eta, jnp.float32),
        jnp.full((16,), lambdas[0], jnp.float32),
    ])
    out = _make_sc_kernel(n)(xf, phi_p, big_p, par)
    return out.reshape(n, 1)


# speedup vs baseline: 1.2103x; 1.1187x over previous
import jax
import jax.numpy as jnp
from jax import lax
from jax.experimental import pallas as pl
from jax.experimental.pallas import tpu as pltpu
from jax.experimental.pallas import tpu_sc as plsc

_NW = 32
_W = 128
_ROWS = 125
_VPR = _W // 16
_MAXK = 8

_PHI_N = 200
_PHI2_N = 100
_PHI_PAD = 208
_PHI2_PAD = 112
_HIDDEN = 3
_SCALE1 = float(_PHI_N - 1)
_SCALE2 = float(_PHI2_N - 1) / 6.0


def _sc_body(x_hbm, phi_hbm, big_hbm, par_hbm, out_hbm,
             xb0, xb1, ob0, ob1, phib, bigb, parb,
             isem0, isem1, osem0, osem1):
    nchunks = x_hbm.shape[0] // _ROWS
    wid = lax.axis_index("s") * 2 + lax.axis_index("c")
    nk = jnp.where(wid < nchunks % _NW, _MAXK, _MAXK - 1)

    pltpu.sync_copy(phi_hbm, phib)
    pltpu.sync_copy(big_hbm, bigb)
    pltpu.sync_copy(par_hbm, parb)
    eta_v = parb[pl.ds(0, 16)]
    lam_v = parb[pl.ds(16, 16)]
    shift = [eta_v * (_SCALE1 * q) for q in range(_HIDDEN)]
    lam2 = lam_v * _SCALE2

    xbufs, obufs = (xb0, xb1), (ob0, ob1)
    isems, osems = (isem0, isem1), (osem0, osem1)

    def start_in(k, b):
        off = (wid + _NW * k) * _ROWS
        pltpu.async_copy(x_hbm.at[pl.ds(off, _ROWS)], xbufs[b], isems[b])

    def wait_in(b):
        pltpu.make_async_copy(
            x_hbm.at[pl.ds(0, _ROWS)], xbufs[b], isems[b]).wait()

    def start_out(k, b):
        off = (wid + _NW * k) * _ROWS
        pltpu.async_copy(obufs[b], out_hbm.at[pl.ds(off, _ROWS)], osems[b])

    def wait_out(b):
        pltpu.make_async_copy(
            obufs[b], out_hbm.at[pl.ds(0, _ROWS)], osems[b]).wait()

    def compute(b):
        xb, ob = xbufs[b], obufs[b]

        @plsc.parallel_loop(0, _ROWS)
        def _row(r):
            for c in range(_VPR):
                v = xb[r, pl.ds(c * 16, 16)]
                acc = None
                for q in range(_HIDDEN):
                    f = jnp.minimum(v * _SCALE1 + shift[q], _SCALE1)
                    ii = jnp.minimum(f.astype(jnp.int32), _PHI_N - 2)
                    t = f - ii.astype(jnp.float32)
                    c0 = plsc.load_gather(phib, [ii])
                    c1 = plsc.load_gather(phib, [ii + 1])
                    phi = c0 + t * (c1 - c0)
                    f2 = jnp.clip(phi * lam2 + (_SCALE2 * (q + 3.0)),
                                  0.0, 6.0 * _SCALE2)
                    jj = jnp.minimum(f2.astype(jnp.int32), _PHI2_N - 2)
                    t2 = f2 - jj.astype(jnp.float32)
                    d0 = plsc.load_gather(bigb, [jj])
                    d1 = plsc.load_gather(bigb, [jj + 1])
                    r_ = d0 + t2 * (d1 - d0)
                    acc = r_ if acc is None else acc + r_
                ob[r, pl.ds(c * 16, 16)] = acc

    start_in(0, 0)
    start_in(1, 1)

    @pl.loop(0, _MAXK - 2, step=2)
    def _pair(k):
        for b in range(2):
            kk = k + b
            wait_in(b)

            @pl.when(kk >= 2)
            def _drain():
                wait_out(b)

            compute(b)
            start_out(kk, b)

            @pl.when(kk + 2 < nk)
            def _next():
                start_in(kk + 2, b)

    wait_in(0)
    wait_out(0)
    compute(0)
    start_out(_MAXK - 2, 0)

    @pl.when(nk == _MAXK)
    def _tail():
        wait_in(1)
        wait_out(1)
        compute(1)
        start_out(_MAXK - 1, 1)

    wait_out(0)
    wait_out(1)


def _make_sc_kernel(rows):
    mesh = plsc.VectorSubcoreMesh(core_axis_name="c", subcore_axis_name="s")
    return pl.kernel(
        _sc_body,
        mesh=mesh,
        compiler_params=pltpu.CompilerParams(
            needs_layout_passes=False, use_tc_tiling_on_sc=False),
        out_type=jax.ShapeDtypeStruct((rows, _W), jnp.float32),
        scratch_types=[
            pltpu.VMEM((_ROWS, _W), jnp.float32),
            pltpu.VMEM((_ROWS, _W), jnp.float32),
            pltpu.VMEM((_ROWS, _W), jnp.float32),
            pltpu.VMEM((_ROWS, _W), jnp.float32),
            pltpu.VMEM((_PHI_PAD,), jnp.float32),
            pltpu.VMEM((_PHI2_PAD,), jnp.float32),
            pltpu.VMEM((32,), jnp.float32),
            pltpu.SemaphoreType.DMA,
            pltpu.SemaphoreType.DMA,
            pltpu.SemaphoreType.DMA,
            pltpu.SemaphoreType.DMA,
        ],
    )


def kernel(x, phi_coeffs, Phi_coeffs, lambdas, eta):
    n = x.shape[0]
    rows = n // _W
    xr = x.reshape(rows, _W)
    phi_p = jnp.zeros((_PHI_PAD,), jnp.float32).at[:_PHI_N].set(phi_coeffs)
    big_p = jnp.zeros((_PHI2_PAD,), jnp.float32).at[:_PHI2_N].set(Phi_coeffs)
    par = jnp.concatenate([
        jnp.full((16,), eta, jnp.float32),
        jnp.full((16,), lambdas[0], jnp.float32),
    ])
    out = _make_sc_kernel(rows)(xr, phi_p, big_p, par)
    return out.reshape(n, 1)


# speedup vs baseline: 1.5078x; 1.2457x over previous
import jax
import jax.numpy as jnp
from jax import lax
from jax.experimental import pallas as pl
from jax.experimental.pallas import tpu as pltpu
from jax.experimental.pallas import tpu_sc as plsc

_NW = 32
_W = 128
_ROWS = 125
_VPR = _W // 16
_MAXK = 8

_PHI_N = 200
_PHI2_N = 100
_PHI_PAD = 208
_PHI2_PAD = 112
_HIDDEN = 3
_F1G = _PHI_N * 32 - 32
_F2G = (_PHI2_N - 1) * 64
_F1PAD = 6384
_F2PAD = 6352


def _sc_body(x_hbm, phi_hbm, big_hbm, par_hbm, out_hbm,
             xb0, xb1, ob0, ob1, phib, bigb, parb, f1b, f2b,
             isem0, isem1, osem0, osem1):
    nchunks = x_hbm.shape[0] // _ROWS
    wid = lax.axis_index("s") * 2 + lax.axis_index("c")
    nk = jnp.where(wid < nchunks % _NW, _MAXK, _MAXK - 1)

    pltpu.sync_copy(phi_hbm, phib)
    pltpu.sync_copy(big_hbm, bigb)
    pltpu.sync_copy(par_hbm, parb)
    eta_v = parb[pl.ds(0, 16)]
    lam_v = parb[pl.ds(16, 16)]
    shift = [eta_v * (float(_F1G) * q) + 0.5 for q in range(_HIDDEN)]
    lam2 = lam_v * (float(_F2G) / 6.0)
    cst2 = [(q + 3.0) * (float(_F2G) / 6.0) + 0.5 for q in range(_HIDDEN)]

    iota = lax.iota(jnp.int32, 16)

    @plsc.parallel_loop(0, _F1PAD // 16)
    def _build1(j):
        g = iota + j * 16
        u = jnp.minimum(g.astype(jnp.float32) * (1.0 / 32.0),
                        float(_PHI_N - 1))
        ii = u.astype(jnp.int32)
        t = u - ii.astype(jnp.float32)
        c0 = plsc.load_gather(phib, [ii])
        c1 = plsc.load_gather(phib, [ii + 1])
        f1b[pl.ds(j * 16, 16)] = c0 + t * (c1 - c0)

    @plsc.parallel_loop(0, _F2PAD // 16)
    def _build2(j):
        g = iota + j * 16
        u = jnp.minimum(g.astype(jnp.float32) * (1.0 / 64.0),
                        float(_PHI2_N - 1))
        ii = u.astype(jnp.int32)
        t = u - ii.astype(jnp.float32)
        d0 = plsc.load_gather(bigb, [ii])
        d1 = plsc.load_gather(bigb, [ii + 1])
        f2b[pl.ds(j * 16, 16)] = d0 + t * (d1 - d0)

    xbufs, obufs = (xb0, xb1), (ob0, ob1)
    isems, osems = (isem0, isem1), (osem0, osem1)

    def start_in(k, b):
        off = (wid + _NW * k) * _ROWS
        pltpu.async_copy(x_hbm.at[pl.ds(off, _ROWS)], xbufs[b], isems[b])

    def wait_in(b):
        pltpu.make_async_copy(
            x_hbm.at[pl.ds(0, _ROWS)], xbufs[b], isems[b]).wait()

    def start_out(k, b):
        off = (wid + _NW * k) * _ROWS
        pltpu.async_copy(obufs[b], out_hbm.at[pl.ds(off, _ROWS)], osems[b])

    def wait_out(b):
        pltpu.make_async_copy(
            obufs[b], out_hbm.at[pl.ds(0, _ROWS)], osems[b]).wait()

    def compute(b):
        xb, ob = xbufs[b], obufs[b]

        @plsc.parallel_loop(0, _ROWS)
        def _row(r):
            for c in range(_VPR):
                v = xb[r, pl.ds(c * 16, 16)]
                acc = None
                for q in range(_HIDDEN):
                    g1 = jnp.minimum(v * float(_F1G) + shift[q], _F1G + 0.49)
                    phi = plsc.load_gather(f1b, [g1.astype(jnp.int32)])
                    g2 = jnp.clip(phi * lam2 + cst2[q], 0.0, _F2G + 0.49)
                    r_ = plsc.load_gather(f2b, [g2.astype(jnp.int32)])
                    acc = r_ if acc is None else acc + r_
                ob[r, pl.ds(c * 16, 16)] = acc

    start_in(0, 0)
    start_in(1, 1)

    @pl.loop(0, _MAXK - 2, step=2)
    def _pair(k):
        for b in range(2):
            kk = k + b
            wait_in(b)

            @pl.when(kk >= 2)
            def _drain():
                wait_out(b)

            compute(b)
            start_out(kk, b)

            @pl.when(kk + 2 < nk)
            def _next():
                start_in(kk + 2, b)

    wait_in(0)
    wait_out(0)
    compute(0)
    start_out(_MAXK - 2, 0)

    @pl.when(nk == _MAXK)
    def _tail():
        wait_in(1)
        wait_out(1)
        compute(1)
        start_out(_MAXK - 1, 1)

    wait_out(0)
    wait_out(1)


def _make_sc_kernel(rows):
    mesh = plsc.VectorSubcoreMesh(core_axis_name="c", subcore_axis_name="s")
    return pl.kernel(
        _sc_body,
        mesh=mesh,
        compiler_params=pltpu.CompilerParams(
            needs_layout_passes=False, use_tc_tiling_on_sc=False),
        out_type=jax.ShapeDtypeStruct((rows, _W), jnp.float32),
        scratch_types=[
            pltpu.VMEM((_ROWS, _W), jnp.float32),
            pltpu.VMEM((_ROWS, _W), jnp.float32),
            pltpu.VMEM((_ROWS, _W), jnp.float32),
            pltpu.VMEM((_ROWS, _W), jnp.float32),
            pltpu.VMEM((_PHI_PAD,), jnp.float32),
            pltpu.VMEM((_PHI2_PAD,), jnp.float32),
            pltpu.VMEM((32,), jnp.float32),
            pltpu.VMEM((_F1PAD,), jnp.float32),
            pltpu.VMEM((_F2PAD,), jnp.float32),
            pltpu.SemaphoreType.DMA,
            pltpu.SemaphoreType.DMA,
            pltpu.SemaphoreType.DMA,
            pltpu.SemaphoreType.DMA,
        ],
    )


def kernel(x, phi_coeffs, Phi_coeffs, lambdas, eta):
    n = x.shape[0]
    rows = n // _W
    xr = x.reshape(rows, _W)
    phi_p = jnp.zeros((_PHI_PAD,), jnp.float32).at[:_PHI_N].set(phi_coeffs)
    big_p = jnp.zeros((_PHI2_PAD,), jnp.float32).at[:_PHI2_N].set(Phi_coeffs)
    par = jnp.concatenate([
        jnp.full((16,), eta, jnp.float32),
        jnp.full((16,), lambdas[0], jnp.float32),
    ])
    out = _make_sc_kernel(rows)(xr, phi_p, big_p, par)
    return out.reshape(n, 1)


# speedup vs baseline: 1.7601x; 1.1674x over previous
import jax
import jax.numpy as jnp
from jax import lax
from jax.experimental import pallas as pl
from jax.experimental.pallas import tpu as pltpu
from jax.experimental.pallas import tpu_sc as plsc

_NW = 32
_W = 128
_ROWS = 125
_VPR = _W // 16
_MAXK = 8

_PHI_N = 200
_PHI2_N = 100
_PHI_PAD = 208
_PHI2_PAD = 112
_HIDDEN = 3
_F1G = _PHI_N * 32 - 32
_F2G = (_PHI2_N - 1) * 64
_F1PAD = 6384
_F2PAD = 6352


def _sc_body(*refs, assemble):
    if assemble:
        (x_hbm, phi_hbm, big_hbm, par_hbm, prev_hbm, out_hbm,
         xb0, xb1, ob0, ob1, phib, bigb, parb, f1b, f2b,
         isem0, isem1, osem0, osem1) = refs
        out_base = prev_hbm.shape[0]
    else:
        (x_hbm, phi_hbm, big_hbm, par_hbm, out_hbm,
         xb0, xb1, ob0, ob1, phib, bigb, parb, f1b, f2b,
         isem0, isem1, osem0, osem1) = refs
        out_base = 0
    nchunks = x_hbm.shape[0] // _ROWS
    maxk = -(-nchunks // _NW)
    assert maxk % 2 == 0 and maxk >= 4
    wid = lax.axis_index("s") * 2 + lax.axis_index("c")
    nk = jnp.where(wid < nchunks % _NW, maxk, maxk - 1)

    pltpu.sync_copy(phi_hbm, phib)
    pltpu.sync_copy(big_hbm, bigb)
    pltpu.sync_copy(par_hbm, parb)
    eta_v = parb[pl.ds(0, 16)]
    lam_v = parb[pl.ds(16, 16)]
    shift = [eta_v * (float(_F1G) * q) + 0.5 for q in range(_HIDDEN)]
    lam2 = lam_v * (float(_F2G) / 6.0)
    cst2 = [(q + 3.0) * (float(_F2G) / 6.0) + 0.5 for q in range(_HIDDEN)]

    iota = lax.iota(jnp.int32, 16)

    @plsc.parallel_loop(0, _F1PAD // 16)
    def _build1(j):
        g = iota + j * 16
        u = jnp.minimum(g.astype(jnp.float32) * (1.0 / 32.0),
                        float(_PHI_N - 1))
        ii = u.astype(jnp.int32)
        t = u - ii.astype(jnp.float32)
        c0 = plsc.load_gather(phib, [ii])
        c1 = plsc.load_gather(phib, [ii + 1])
        f1b[pl.ds(j * 16, 16)] = c0 + t * (c1 - c0)

    @plsc.parallel_loop(0, _F2PAD // 16)
    def _build2(j):
        g = iota + j * 16
        u = jnp.minimum(g.astype(jnp.float32) * (1.0 / 64.0),
                        float(_PHI2_N - 1))
        ii = u.astype(jnp.int32)
        t = u - ii.astype(jnp.float32)
        d0 = plsc.load_gather(bigb, [ii])
        d1 = plsc.load_gather(bigb, [ii + 1])
        f2b[pl.ds(j * 16, 16)] = d0 + t * (d1 - d0)

    xbufs, obufs = (xb0, xb1), (ob0, ob1)
    isems, osems = (isem0, isem1), (osem0, osem1)

    def start_in(k, b):
        off = (wid + _NW * k) * _ROWS
        pltpu.async_copy(x_hbm.at[pl.ds(off, _ROWS)], xbufs[b], isems[b])

    def wait_in(b):
        pltpu.make_async_copy(
            x_hbm.at[pl.ds(0, _ROWS)], xbufs[b], isems[b]).wait()

    def start_out(k, b):
        off = out_base + (wid + _NW * k) * _ROWS
        pltpu.async_copy(obufs[b], out_hbm.at[pl.ds(off, _ROWS)], osems[b])

    def wait_out(b):
        pltpu.make_async_copy(
            obufs[b], out_hbm.at[pl.ds(0, _ROWS)], osems[b]).wait()

    def compute(b):
        xb, ob = xbufs[b], obufs[b]

        @plsc.parallel_loop(0, _ROWS)
        def _row(r):
            for c in range(_VPR):
                v = xb[r, pl.ds(c * 16, 16)]
                acc = None
                for q in range(_HIDDEN):
                    g1 = jnp.minimum(v * float(_F1G) + shift[q], _F1G + 0.49)
                    phi = plsc.load_gather(f1b, [g1.astype(jnp.int32)])
                    g2 = jnp.clip(phi * lam2 + cst2[q], 0.0, _F2G + 0.49)
                    r_ = plsc.load_gather(f2b, [g2.astype(jnp.int32)])
                    acc = r_ if acc is None else acc + r_
                ob[r, pl.ds(c * 16, 16)] = acc

    start_in(0, 0)
    start_in(1, 1)

    @pl.loop(0, maxk - 2, step=2)
    def _pair(k):
        for b in range(2):
            kk = k + b
            wait_in(b)

            @pl.when(kk >= 2)
            def _drain():
                wait_out(b)

            compute(b)
            start_out(kk, b)

            @pl.when(kk + 2 < nk)
            def _next():
                start_in(kk + 2, b)

    wait_in(0)
    wait_out(0)
    compute(0)
    start_out(maxk - 2, 0)

    @pl.when(nk == maxk)
    def _tail():
        wait_in(1)
        wait_out(1)
        compute(1)
        start_out(maxk - 1, 1)

    wait_out(0)
    wait_out(1)

    if assemble:
        achunks = prev_hbm.shape[0] // _ROWS
        amax = -(-achunks // _NW)
        ank = jnp.where(wid < achunks % _NW, amax, amax - 1)

        def acopy(k):
            off = (wid + _NW * k) * _ROWS
            pltpu.sync_copy(prev_hbm.at[pl.ds(off, _ROWS)], xb0)
            pltpu.sync_copy(xb0, out_hbm.at[pl.ds(off, _ROWS)])

        @pl.loop(0, amax - 1)
        def _acopy(k):
            acopy(k)

        @pl.when(ank == amax)
        def _alast():
            acopy(amax - 1)


def _make_sc_kernel(rows, assemble=False):
    import functools
    mesh = plsc.VectorSubcoreMesh(core_axis_name="c", subcore_axis_name="s")
    return pl.kernel(
        functools.partial(_sc_body, assemble=assemble),
        mesh=mesh,
        compiler_params=pltpu.CompilerParams(
            needs_layout_passes=False, use_tc_tiling_on_sc=False),
        out_type=jax.ShapeDtypeStruct((rows, _W), jnp.float32),
        scratch_types=[
            pltpu.VMEM((_ROWS, _W), jnp.float32),
            pltpu.VMEM((_ROWS, _W), jnp.float32),
            pltpu.VMEM((_ROWS, _W), jnp.float32),
            pltpu.VMEM((_ROWS, _W), jnp.float32),
            pltpu.VMEM((_PHI_PAD,), jnp.float32),
            pltpu.VMEM((_PHI2_PAD,), jnp.float32),
            pltpu.VMEM((32,), jnp.float32),
            pltpu.VMEM((_F1PAD,), jnp.float32),
            pltpu.VMEM((_F2PAD,), jnp.float32),
            pltpu.SemaphoreType.DMA,
            pltpu.SemaphoreType.DMA,
            pltpu.SemaphoreType.DMA,
            pltpu.SemaphoreType.DMA,
        ],
    )


def kernel(x, phi_coeffs, Phi_coeffs, lambdas, eta):
    n = x.shape[0]
    rows = n // _W
    rows_a = rows // 2
    na = rows_a * _W
    phi_p = jnp.zeros((_PHI_PAD,), jnp.float32).at[:_PHI_N].set(phi_coeffs)
    big_p = jnp.zeros((_PHI2_PAD,), jnp.float32).at[:_PHI2_N].set(Phi_coeffs)
    par = jnp.concatenate([
        jnp.full((16,), eta, jnp.float32),
        jnp.full((16,), lambdas[0], jnp.float32),
    ])
    xa = x[:na].reshape(rows_a, _W)
    xb = x[na:].reshape(rows - rows_a, _W)
    out_a = _make_sc_kernel(rows_a)(xa, phi_p, big_p, par)
    out = _make_sc_kernel(rows, assemble=True)(xb, phi_p, big_p, par, out_a)
    return out.reshape(n, 1)
